# baseline (speedup 1.0000x reference)
import functools

import numpy as np
import jax
import jax.numpy as jnp
from jax import lax
from jax.experimental import pallas as pl
from jax.experimental.pallas import tpu as pltpu
from jax.experimental.pallas import tpu_sc as plsc

B, L, V, E, O = 4096, 50, 100000, 128, 10000

NCH = 4
B_CH = B // NCH

NC = 2
NS = 16
NW = NC * NS
CH_ROWS = B_CH * L
ROWS_PER_W = CH_ROWS // NW
CHUNK = 64
K_INFLIGHT = 5
OUTER = ROWS_PER_W // (CHUNK * K_INFLIGHT)

_sc_mesh = plsc.VectorSubcoreMesh(core_axis_name="c", subcore_axis_name="s")


@functools.partial(
    pl.kernel,
    mesh=_sc_mesh,
    out_type=jax.ShapeDtypeStruct((CH_ROWS, E), jnp.float32),
    scratch_types=[
        pltpu.VMEM((ROWS_PER_W,), jnp.int32),
        pltpu.VMEM((CHUNK * K_INFLIGHT, E), jnp.float32),
        pltpu.SemaphoreType.DMA,
    ],
)
def _sc_gather(idx_hbm, table_hbm, out_hbm, idx_v, rows_v, sem):
    wid = lax.axis_index("s") * NC + lax.axis_index("c")
    base = wid * ROWS_PER_W
    pltpu.sync_copy(idx_hbm.at[pl.ds(base, ROWS_PER_W)], idx_v)
    for outer in range(OUTER):
        o0 = outer * CHUNK * K_INFLIGHT
        copies = []
        for j in range(K_INFLIGHT):
            copies.append(
                pltpu.async_copy(
                    table_hbm.at[idx_v.at[pl.ds(o0 + j * CHUNK, CHUNK)]],
                    rows_v.at[pl.ds(j * CHUNK, CHUNK)],
                    sem,
                )
            )
        for cp in copies:
            cp.wait()
        pltpu.sync_copy(rows_v, out_hbm.at[pl.ds(base + o0, CHUNK * K_INFLIGHT)])


BBLK = 256
STEPS_CH = B_CH // BBLK
GAMMA = np.float32(np.float32(0.9) * np.float32(L - 1) - np.float32(44.0))


def _tc_impl(g_ref, wt_ref, eb_ref, b_ref, out_ref, wt_vmem, b_vmem, sem):
    @pl.when(pl.program_id(0) == 0)
    def _load_w():
        cp = pltpu.make_async_copy(wt_ref, wt_vmem, sem)
        cp.start()
        cp.wait()
        cpb = pltpu.make_async_copy(b_ref, b_vmem, sem)
        cpb.start()
        cpb.wait()

    neg_inf = jnp.float32(-jnp.inf)
    top = [jnp.full((BBLK, E), neg_inf, jnp.float32) for _ in range(6)]
    for l in range(L):
        x = g_ref[l]
        for k in range(6):
            hi = jnp.maximum(top[k], x)
            x = jnp.minimum(top[k], x)
            top[k] = hi
    qs = top[5] + GAMMA * (top[4] - top[5])
    acc = jnp.zeros((BBLK, E), jnp.float32)
    for l in range(L):
        x = g_ref[l]
        acc = acc + jnp.where(x >= qs, x, 0.0)
    s = acc + eb_ref[...]
    out_ref[...] = (
        jnp.dot(s, wt_vmem[...], preferred_element_type=jnp.float32)
        + b_vmem[...]
    )


def _tc_body_first(g_ref, wt_ref, eb_ref, b_ref, out_ref,
                   wt_vmem, b_vmem, sem):
    _tc_impl(g_ref, wt_ref, eb_ref, b_ref, out_ref, wt_vmem, b_vmem, sem)


def _tc_body_rest(g_ref, wt_ref, eb_ref, b_ref, prev_ref, out_ref,
                  wt_vmem, b_vmem, sem):
    del prev_ref
    _tc_impl(g_ref, wt_ref, eb_ref, b_ref, out_ref, wt_vmem, b_vmem, sem)


_SCRATCH = [
    pltpu.VMEM((E, O), jnp.float32),
    pltpu.VMEM((1, O), jnp.float32),
    pltpu.SemaphoreType.DMA,
]

_BASE_SPECS = [
    pl.BlockSpec((L, BBLK, E), lambda i: (0, i, 0)),
    pl.BlockSpec(memory_space=pltpu.MemorySpace.HBM),
    pl.BlockSpec((1, E), lambda i: (0, 0)),
    pl.BlockSpec(memory_space=pltpu.MemorySpace.HBM),
]


def _make_tc_call(chunk):
    first = chunk == 0
    return pl.pallas_call(
        _tc_body_first if first else _tc_body_rest,
        grid=(STEPS_CH,),
        in_specs=_BASE_SPECS if first else (
            _BASE_SPECS + [pl.BlockSpec(memory_space=pltpu.MemorySpace.HBM)]
        ),
        out_specs=pl.BlockSpec(
            (BBLK, O), lambda i, c=chunk: (c * STEPS_CH + i, 0)
        ),
        out_shape=jax.ShapeDtypeStruct((B, O), jnp.float32),
        scratch_shapes=_SCRATCH,
        input_output_aliases={} if first else {4: 0},
    )


_tc_calls = [_make_tc_call(c) for c in range(NCH)]


def kernel(tokens, emb_table, emb_bias, W, b):
    idx_t = tokens.astype(jnp.int32).T
    wt = W.T
    eb = emb_bias.reshape(1, E)
    bb = b.reshape(1, O)
    gathered = []
    for c in range(NCH):
        idx_c = idx_t[:, c * B_CH:(c + 1) * B_CH].reshape(-1)
        gathered.append(_sc_gather(idx_c, emb_table))
    out = _tc_calls[0](gathered[0].reshape(L, B_CH, E), wt, eb, bb)
    for c in range(1, NCH):
        g3 = gathered[c].reshape(L, B_CH, E)
        out = _tc_calls[c](g3, wt, eb, bb, out)
    return out


# speedup vs baseline: 1.0289x; 1.0289x over previous
import functools

import numpy as np
import jax
import jax.numpy as jnp
from jax import lax
from jax.experimental import pallas as pl
from jax.experimental.pallas import tpu as pltpu
from jax.experimental.pallas import tpu_sc as plsc

B, L, V, E, O = 4096, 50, 100000, 128, 10000

NCH = 2
B_CH = B // NCH

NC = 2
NS = 16
NW = NC * NS
CH_ROWS = B_CH * L
ROWS_PER_W = CH_ROWS // NW
CHUNK = 128
K_INFLIGHT = 5
OUTER = ROWS_PER_W // (CHUNK * K_INFLIGHT)

_sc_mesh = plsc.VectorSubcoreMesh(core_axis_name="c", subcore_axis_name="s")


@functools.partial(
    pl.kernel,
    mesh=_sc_mesh,
    out_type=jax.ShapeDtypeStruct((CH_ROWS, E), jnp.float32),
    scratch_types=[
        pltpu.VMEM((ROWS_PER_W,), jnp.int32),
        pltpu.VMEM((CHUNK * K_INFLIGHT, E), jnp.float32),
        pltpu.SemaphoreType.DMA,
    ],
)
def _sc_gather(idx_hbm, table_hbm, out_hbm, idx_v, rows_v, sem):
    wid = lax.axis_index("s") * NC + lax.axis_index("c")
    base = wid * ROWS_PER_W
    pltpu.sync_copy(idx_hbm.at[pl.ds(base, ROWS_PER_W)], idx_v)
    for outer in range(OUTER):
        o0 = outer * CHUNK * K_INFLIGHT
        copies = []
        for j in range(K_INFLIGHT):
            copies.append(
                pltpu.async_copy(
                    table_hbm.at[idx_v.at[pl.ds(o0 + j * CHUNK, CHUNK)]],
                    rows_v.at[pl.ds(j * CHUNK, CHUNK)],
                    sem,
                )
            )
        for cp in copies:
            cp.wait()
        pltpu.sync_copy(rows_v, out_hbm.at[pl.ds(base + o0, CHUNK * K_INFLIGHT)])


BBLK = 256
STEPS_CH = B_CH // BBLK
GAMMA = np.float32(np.float32(0.9) * np.float32(L - 1) - np.float32(44.0))


def _tc_impl(g_ref, wt_ref, eb_ref, b_ref, out_ref, wt_vmem, b_vmem, sem):
    @pl.when(pl.program_id(0) == 0)
    def _load_w():
        cp = pltpu.make_async_copy(wt_ref, wt_vmem, sem)
        cp.start()
        cp.wait()
        cpb = pltpu.make_async_copy(b_ref, b_vmem, sem)
        cpb.start()
        cpb.wait()

    neg_inf = jnp.float32(-jnp.inf)
    top = [jnp.full((BBLK, E), neg_inf, jnp.float32) for _ in range(6)]
    for l in range(L):
        x = g_ref[l]
        for k in range(6):
            hi = jnp.maximum(top[k], x)
            x = jnp.minimum(top[k], x)
            top[k] = hi
    qs = top[5] + GAMMA * (top[4] - top[5])
    acc = jnp.zeros((BBLK, E), jnp.float32)
    for l in range(L):
        x = g_ref[l]
        acc = acc + jnp.where(x >= qs, x, 0.0)
    s = acc + eb_ref[...]
    out_ref[...] = (
        jnp.dot(s, wt_vmem[...], preferred_element_type=jnp.float32)
        + b_vmem[...]
    )


def _tc_body_first(g_ref, wt_ref, eb_ref, b_ref, out_ref,
                   wt_vmem, b_vmem, sem):
    _tc_impl(g_ref, wt_ref, eb_ref, b_ref, out_ref, wt_vmem, b_vmem, sem)


def _tc_body_rest(g_ref, wt_ref, eb_ref, b_ref, prev_ref, out_ref,
                  wt_vmem, b_vmem, sem):
    del prev_ref
    _tc_impl(g_ref, wt_ref, eb_ref, b_ref, out_ref, wt_vmem, b_vmem, sem)


_SCRATCH = [
    pltpu.VMEM((E, O), jnp.float32),
    pltpu.VMEM((1, O), jnp.float32),
    pltpu.SemaphoreType.DMA,
]

_BASE_SPECS = [
    pl.BlockSpec((L, BBLK, E), lambda i: (0, i, 0)),
    pl.BlockSpec(memory_space=pltpu.MemorySpace.HBM),
    pl.BlockSpec((1, E), lambda i: (0, 0)),
    pl.BlockSpec(memory_space=pltpu.MemorySpace.HBM),
]


def _make_tc_call(chunk):
    first = chunk == 0
    return pl.pallas_call(
        _tc_body_first if first else _tc_body_rest,
        grid=(STEPS_CH,),
        in_specs=_BASE_SPECS if first else (
            _BASE_SPECS + [pl.BlockSpec(memory_space=pltpu.MemorySpace.HBM)]
        ),
        out_specs=pl.BlockSpec(
            (BBLK, O), lambda i, c=chunk: (c * STEPS_CH + i, 0)
        ),
        out_shape=jax.ShapeDtypeStruct((B, O), jnp.float32),
        scratch_shapes=_SCRATCH,
        input_output_aliases={} if first else {4: 0},
    )


_tc_calls = [_make_tc_call(c) for c in range(NCH)]


def kernel(tokens, emb_table, emb_bias, W, b):
    idx_t = tokens.astype(jnp.int32).T
    wt = W.T
    eb = emb_bias.reshape(1, E)
    bb = b.reshape(1, O)
    gathered = []
    for c in range(NCH):
        idx_c = idx_t[:, c * B_CH:(c + 1) * B_CH].reshape(-1)
        gathered.append(_sc_gather(idx_c, emb_table))
    out = _tc_calls[0](gathered[0].reshape(L, B_CH, E), wt, eb, bb)
    for c in range(1, NCH):
        g3 = gathered[c].reshape(L, B_CH, E)
        out = _tc_calls[c](g3, wt, eb, bb, out)
    return out
